# Initial kernel scaffold; baseline (speedup 1.0000x reference)
import jax
import jax.numpy as jnp
from jax.experimental import pallas as pl


def kernel(species, rand_encoding):
    raise NotImplementedError("write your pallas kernel here")


# speedup vs baseline: 1.2934x; 1.2934x over previous
import functools

import jax
import jax.numpy as jnp
from jax import lax
from jax.experimental import pallas as pl
from jax.experimental.pallas import tpu as pltpu
from jax.experimental.pallas import tpu_sc as plsc

B = 2_000_000
ZMAXPAD = 52
DIM = 64

NC = 2
NS = 16
NW = NC * NS

CHUNK = 104
PER_W = 62_504
N_CHUNKS = PER_W // CHUNK


def _body(species_hbm, table_hbm, out_hbm, idx_v, rows_v, sem):
    wid = lax.axis_index("s") * NC + lax.axis_index("c")
    base = (wid * (B // NW + 4)) // 8 * 8
    base = jnp.minimum(base, B - PER_W)

    def step(g, _):
        off = base + g * CHUNK
        pltpu.sync_copy(species_hbm.at[pl.ds(off, CHUNK)], idx_v)
        pltpu.async_copy(table_hbm.at[idx_v], rows_v, sem).wait()
        pltpu.sync_copy(rows_v, out_hbm.at[pl.ds(off, CHUNK)])
        return _

    lax.fori_loop(0, N_CHUNKS, step, None)


@jax.jit
def _gather(species, table):
    kern = pl.kernel(
        _body,
        out_type=jax.ShapeDtypeStruct((B, DIM), jnp.float32),
        mesh=plsc.VectorSubcoreMesh(core_axis_name="c", subcore_axis_name="s"),
        scratch_types=[
            pltpu.VMEM((CHUNK,), jnp.int32),
            pltpu.VMEM((CHUNK, DIM), jnp.float32),
            pltpu.SemaphoreType.DMA,
        ],
        compiler_params=pltpu.CompilerParams(use_tc_tiling_on_sc=False),
    )
    return kern(species, table)


def kernel(species, rand_encoding):
    return _gather(species, rand_encoding)


# speedup vs baseline: 1.3239x; 1.0235x over previous
import jax
import jax.numpy as jnp
from jax import lax
from jax.experimental import pallas as pl
from jax.experimental.pallas import tpu as pltpu
from jax.experimental.pallas import tpu_sc as plsc

B = 2_000_000
ZMAXPAD = 52
DIM = 64

NC = 2
NS = 16
NW = NC * NS

CHUNK = 128
K = 4
MACRO = K * CHUNK
N_MACRO = 124
PER_W = N_MACRO * MACRO


def _body(species_hbm, table_hbm, out_hbm,
          idx0, idx1, rows0, rows1,
          sem_i0, sem_i1, sem_g0, sem_g1, sem_s0, sem_s1):
    idx_b = (idx0, idx1)
    rows_b = (rows0, rows1)
    sem_i = (sem_i0, sem_i1)
    sem_g = (sem_g0, sem_g1)
    sem_s = (sem_s0, sem_s1)

    wid = lax.axis_index("s") * NC + lax.axis_index("c")
    base = (wid * (B // NW + 4)) // 8 * 8
    base = jnp.minimum(base, B - PER_W)

    def start_idx(m, b):
        pltpu.async_copy(species_hbm.at[pl.ds(base + m * MACRO, MACRO)],
                         idx_b[b], sem_i[b])

    start_idx(0, 0)

    def outer(o, _):
        for b in (0, 1):
            m = 2 * o + b

            @pl.when(m + 1 < N_MACRO)
            def _():
                start_idx(m + 1, 1 - b)

            pltpu.make_async_copy(species_hbm.at[pl.ds(0, MACRO)],
                                  idx_b[b], sem_i[b]).wait()

            @pl.when(m >= 2)
            def _():
                pltpu.make_async_copy(rows_b[b],
                                      out_hbm.at[pl.ds(0, MACRO)],
                                      sem_s[b]).wait()

            for j in range(K):
                pltpu.async_copy(
                    table_hbm.at[idx_b[b].at[pl.ds(j * CHUNK, CHUNK)]],
                    rows_b[b].at[pl.ds(j * CHUNK, CHUNK)],
                    sem_g[b])
            for j in range(K):
                pltpu.make_async_copy(
                    table_hbm.at[idx_b[b].at[pl.ds(j * CHUNK, CHUNK)]],
                    rows_b[b].at[pl.ds(j * CHUNK, CHUNK)],
                    sem_g[b]).wait()

            pltpu.async_copy(rows_b[b],
                             out_hbm.at[pl.ds(base + m * MACRO, MACRO)],
                             sem_s[b])
        return _

    lax.fori_loop(0, N_MACRO // 2, outer, None)

    for b in (0, 1):
        pltpu.make_async_copy(rows_b[b], out_hbm.at[pl.ds(0, MACRO)],
                              sem_s[b]).wait()


@jax.jit
def _gather(species, table):
    kern = pl.kernel(
        _body,
        out_type=jax.ShapeDtypeStruct((B, DIM), jnp.float32),
        mesh=plsc.VectorSubcoreMesh(core_axis_name="c", subcore_axis_name="s"),
        scratch_types=[
            pltpu.VMEM((MACRO,), jnp.int32),
            pltpu.VMEM((MACRO,), jnp.int32),
            pltpu.VMEM((MACRO, DIM), jnp.float32),
            pltpu.VMEM((MACRO, DIM), jnp.float32),
            pltpu.SemaphoreType.DMA,
            pltpu.SemaphoreType.DMA,
            pltpu.SemaphoreType.DMA,
            pltpu.SemaphoreType.DMA,
            pltpu.SemaphoreType.DMA,
            pltpu.SemaphoreType.DMA,
        ],
        compiler_params=pltpu.CompilerParams(use_tc_tiling_on_sc=False),
    )
    return kern(species, table)


def kernel(species, rand_encoding):
    return _gather(species, rand_encoding)


# speedup vs baseline: 3.3115x; 2.5014x over previous
import jax
import jax.numpy as jnp
from jax import lax
from jax.experimental import pallas as pl
from jax.experimental.pallas import tpu as pltpu
from jax.experimental.pallas import tpu_sc as plsc

B = 2_000_000
ZMAXPAD = 52
DIM = 64

NC = 2
NS = 16
NW = NC * NS

CHUNK = 128
K = 4
MACRO = K * CHUNK
N_MACRO = 124
PER_W = N_MACRO * MACRO


def _body(species_hbm, table_hbm, out_hbm,
          idx0, idx1, rows0, rows1, table_sh,
          sem_i0, sem_i1, sem_g0, sem_g1, sem_s0, sem_s1):
    idx_b = (idx0, idx1)
    rows_b = (rows0, rows1)
    sem_i = (sem_i0, sem_i1)
    sem_g = (sem_g0, sem_g1)
    sem_s = (sem_s0, sem_s1)

    @pl.when(lax.axis_index("s") == 0)
    def _():
        pltpu.sync_copy(table_hbm, table_sh)

    plsc.subcore_barrier()

    wid = lax.axis_index("s") * NC + lax.axis_index("c")
    base = (wid * (B // NW + 4)) // 8 * 8
    base = jnp.minimum(base, B - PER_W)

    def start_idx(m, b):
        pltpu.async_copy(species_hbm.at[pl.ds(base + m * MACRO, MACRO)],
                         idx_b[b], sem_i[b])

    start_idx(0, 0)

    def outer(o, _):
        for b in (0, 1):
            m = 2 * o + b

            @pl.when(m + 1 < N_MACRO)
            def _():
                start_idx(m + 1, 1 - b)

            pltpu.make_async_copy(species_hbm.at[pl.ds(0, MACRO)],
                                  idx_b[b], sem_i[b]).wait()

            @pl.when(m >= 2)
            def _():
                pltpu.make_async_copy(rows_b[b],
                                      out_hbm.at[pl.ds(0, MACRO)],
                                      sem_s[b]).wait()

            for j in range(K):
                pltpu.async_copy(
                    table_sh.at[idx_b[b].at[pl.ds(j * CHUNK, CHUNK)]],
                    rows_b[b].at[pl.ds(j * CHUNK, CHUNK)],
                    sem_g[b])
            for j in range(K):
                pltpu.make_async_copy(
                    table_sh.at[idx_b[b].at[pl.ds(j * CHUNK, CHUNK)]],
                    rows_b[b].at[pl.ds(j * CHUNK, CHUNK)],
                    sem_g[b]).wait()

            pltpu.async_copy(rows_b[b],
                             out_hbm.at[pl.ds(base + m * MACRO, MACRO)],
                             sem_s[b])
        return _

    lax.fori_loop(0, N_MACRO // 2, outer, None)

    for b in (0, 1):
        pltpu.make_async_copy(rows_b[b], out_hbm.at[pl.ds(0, MACRO)],
                              sem_s[b]).wait()


@jax.jit
def _gather(species, table):
    kern = pl.kernel(
        _body,
        out_type=jax.ShapeDtypeStruct((B, DIM), jnp.float32),
        mesh=plsc.VectorSubcoreMesh(core_axis_name="c", subcore_axis_name="s"),
        scratch_types=[
            pltpu.VMEM((MACRO,), jnp.int32),
            pltpu.VMEM((MACRO,), jnp.int32),
            pltpu.VMEM((MACRO, DIM), jnp.float32),
            pltpu.VMEM((MACRO, DIM), jnp.float32),
            pltpu.VMEM_SHARED((ZMAXPAD, DIM), jnp.float32),
            pltpu.SemaphoreType.DMA,
            pltpu.SemaphoreType.DMA,
            pltpu.SemaphoreType.DMA,
            pltpu.SemaphoreType.DMA,
            pltpu.SemaphoreType.DMA,
            pltpu.SemaphoreType.DMA,
        ],
        compiler_params=pltpu.CompilerParams(use_tc_tiling_on_sc=False),
    )
    return kern(species, table)


def kernel(species, rand_encoding):
    return _gather(species, rand_encoding)


# speedup vs baseline: 4.4821x; 1.3535x over previous
import jax
import jax.numpy as jnp
from jax import lax
from jax.experimental import pallas as pl
from jax.experimental.pallas import tpu as pltpu
from jax.experimental.pallas import tpu_sc as plsc

B = 2_000_000
ZMAXPAD = 52
DIM = 64

NC = 2
NS = 16
NW = NC * NS

CHUNK = 128
K = 1
MACRO = K * CHUNK
N_MACRO = 496
PER_W = N_MACRO * MACRO


def _body(species_hbm, table_hbm, out_hbm,
          idx0, idx1, rows0, rows1, table_sh,
          sem_i0, sem_i1, sem_g0, sem_g1, sem_s0, sem_s1):
    idx_b = (idx0, idx1)
    rows_b = (rows0, rows1)
    sem_i = (sem_i0, sem_i1)
    sem_g = (sem_g0, sem_g1)
    sem_s = (sem_s0, sem_s1)

    @pl.when(lax.axis_index("s") == 0)
    def _():
        pltpu.sync_copy(table_hbm, table_sh)

    plsc.subcore_barrier()

    wid = lax.axis_index("s") * NC + lax.axis_index("c")
    base = (wid * (B // NW + 4)) // 8 * 8
    base = jnp.minimum(base, B - PER_W)

    def start_idx(m, b):
        pltpu.async_copy(species_hbm.at[pl.ds(base + m * MACRO, MACRO)],
                         idx_b[b], sem_i[b])

    start_idx(0, 0)

    def outer(o, _):
        for b in (0, 1):
            m = 2 * o + b

            @pl.when(m + 1 < N_MACRO)
            def _():
                start_idx(m + 1, 1 - b)

            pltpu.make_async_copy(species_hbm.at[pl.ds(0, MACRO)],
                                  idx_b[b], sem_i[b]).wait()

            @pl.when(m >= 2)
            def _():
                pltpu.make_async_copy(rows_b[b],
                                      out_hbm.at[pl.ds(0, MACRO)],
                                      sem_s[b]).wait()

            for j in range(K):
                pltpu.async_copy(
                    table_sh.at[idx_b[b].at[pl.ds(j * CHUNK, CHUNK)]],
                    rows_b[b].at[pl.ds(j * CHUNK, CHUNK)],
                    sem_g[b])
            for j in range(K):
                pltpu.make_async_copy(
                    table_sh.at[idx_b[b].at[pl.ds(j * CHUNK, CHUNK)]],
                    rows_b[b].at[pl.ds(j * CHUNK, CHUNK)],
                    sem_g[b]).wait()

            pltpu.async_copy(rows_b[b],
                             out_hbm.at[pl.ds(base + m * MACRO, MACRO)],
                             sem_s[b])
        return _

    lax.fori_loop(0, N_MACRO // 2, outer, None)

    for b in (0, 1):
        pltpu.make_async_copy(rows_b[b], out_hbm.at[pl.ds(0, MACRO)],
                              sem_s[b]).wait()


@jax.jit
def _gather(species, table):
    kern = pl.kernel(
        _body,
        out_type=jax.ShapeDtypeStruct((B, DIM), jnp.float32),
        mesh=plsc.VectorSubcoreMesh(core_axis_name="c", subcore_axis_name="s"),
        scratch_types=[
            pltpu.VMEM((MACRO,), jnp.int32),
            pltpu.VMEM((MACRO,), jnp.int32),
            pltpu.VMEM((MACRO, DIM), jnp.float32),
            pltpu.VMEM((MACRO, DIM), jnp.float32),
            pltpu.VMEM_SHARED((ZMAXPAD, DIM), jnp.float32),
            pltpu.SemaphoreType.DMA,
            pltpu.SemaphoreType.DMA,
            pltpu.SemaphoreType.DMA,
            pltpu.SemaphoreType.DMA,
            pltpu.SemaphoreType.DMA,
            pltpu.SemaphoreType.DMA,
        ],
        compiler_params=pltpu.CompilerParams(use_tc_tiling_on_sc=True),
    )
    return kern(species, table)


def kernel(species, rand_encoding):
    return _gather(species, rand_encoding)


# speedup vs baseline: 4.4823x; 1.0000x over previous
import jax
import jax.numpy as jnp
from jax import lax
from jax.experimental import pallas as pl
from jax.experimental.pallas import tpu as pltpu
from jax.experimental.pallas import tpu_sc as plsc

B = 2_000_000
ZMAXPAD = 52
DIM = 64

NC = 2
NS = 16
NW = NC * NS

MACRO = 128
R = 4
LAG = 2
N_MACRO = 496
PER_W = N_MACRO * MACRO


def _body(species_hbm, table_hbm, out_hbm,
          idx0, idx1, idx2, idx3, rows0, rows1, rows2, rows3, table_sh,
          si0, si1, si2, si3, sg0, sg1, sg2, sg3, ss0, ss1, ss2, ss3):
    idx_b = (idx0, idx1, idx2, idx3)
    rows_b = (rows0, rows1, rows2, rows3)
    sem_i = (si0, si1, si2, si3)
    sem_g = (sg0, sg1, sg2, sg3)
    sem_s = (ss0, ss1, ss2, ss3)

    @pl.when(lax.axis_index("s") == 0)
    def _():
        pltpu.sync_copy(table_hbm, table_sh)

    plsc.subcore_barrier()

    wid = lax.axis_index("s") * NC + lax.axis_index("c")
    base = (wid * (B // NW + 4)) // 8 * 8
    base = jnp.minimum(base, B - PER_W)

    def start_idx(m, b):
        pltpu.async_copy(species_hbm.at[pl.ds(base + m * MACRO, MACRO)],
                         idx_b[b], sem_i[b])

    def wait_idx(b):
        pltpu.make_async_copy(species_hbm.at[pl.ds(0, MACRO)],
                              idx_b[b], sem_i[b]).wait()

    def start_gather(b):
        pltpu.async_copy(table_sh.at[idx_b[b]], rows_b[b], sem_g[b])

    def wait_gather(b):
        pltpu.make_async_copy(table_sh.at[idx_b[b]], rows_b[b],
                              sem_g[b]).wait()

    def start_store(m, b):
        pltpu.async_copy(rows_b[b],
                         out_hbm.at[pl.ds(base + m * MACRO, MACRO)],
                         sem_s[b])

    def wait_store(b):
        pltpu.make_async_copy(rows_b[b], out_hbm.at[pl.ds(0, MACRO)],
                              sem_s[b]).wait()

    for b in range(R):
        start_idx(b, b)

    def outer(o, _):
        for q in range(R):
            b = q
            m = R * o + q

            wait_idx(b)

            @pl.when(m >= R)
            def _():
                wait_store(b)

            start_gather(b)

            d = m - LAG
            bd = (q + R - LAG) % R

            @pl.when(m >= LAG)
            def _():
                wait_gather(bd)
                start_store(d, bd)

            @pl.when((m >= LAG) & (m + LAG < N_MACRO))
            def _():
                start_idx(m + LAG, bd)
        return _

    lax.fori_loop(0, N_MACRO // R, outer, None)

    for m in range(N_MACRO - LAG, N_MACRO):
        bd = m % R
        wait_gather(bd)
        start_store(m, bd)
    for b in range(R):
        wait_store(b)


@jax.jit
def _gather(species, table):
    kern = pl.kernel(
        _body,
        out_type=jax.ShapeDtypeStruct((B, DIM), jnp.float32),
        mesh=plsc.VectorSubcoreMesh(core_axis_name="c", subcore_axis_name="s"),
        scratch_types=[
            pltpu.VMEM((MACRO,), jnp.int32),
            pltpu.VMEM((MACRO,), jnp.int32),
            pltpu.VMEM((MACRO,), jnp.int32),
            pltpu.VMEM((MACRO,), jnp.int32),
            pltpu.VMEM((MACRO, DIM), jnp.float32),
            pltpu.VMEM((MACRO, DIM), jnp.float32),
            pltpu.VMEM((MACRO, DIM), jnp.float32),
            pltpu.VMEM((MACRO, DIM), jnp.float32),
            pltpu.VMEM_SHARED((ZMAXPAD, DIM), jnp.float32),
            pltpu.SemaphoreType.DMA,
            pltpu.SemaphoreType.DMA,
            pltpu.SemaphoreType.DMA,
            pltpu.SemaphoreType.DMA,
            pltpu.SemaphoreType.DMA,
            pltpu.SemaphoreType.DMA,
            pltpu.SemaphoreType.DMA,
            pltpu.SemaphoreType.DMA,
            pltpu.SemaphoreType.DMA,
            pltpu.SemaphoreType.DMA,
            pltpu.SemaphoreType.DMA,
            pltpu.SemaphoreType.DMA,
        ],
        compiler_params=pltpu.CompilerParams(use_tc_tiling_on_sc=True),
    )
    return kern(species, table)


def kernel(species, rand_encoding):
    return _gather(species, rand_encoding)


# speedup vs baseline: 4.5508x; 1.0153x over previous
import jax
import jax.numpy as jnp
from jax.experimental import pallas as pl

B = 2_000_000
ZMAXPAD = 52
DIM = 64

BLK = 16384


def _body(idx_ref, table_ref, out_ref):
    idx = idx_ref[...]
    tbl = jnp.pad(table_ref[...], ((0, DIM - ZMAXPAD), (0, 0)))
    oh = (idx[:, None]
          == jax.lax.broadcasted_iota(jnp.int32, (1, DIM), 1)
          ).astype(jnp.float32)
    out_ref[...] = jnp.dot(oh, tbl, preferred_element_type=jnp.float32)


@jax.jit
def _encode(species, table):
    return pl.pallas_call(
        _body,
        grid=(pl.cdiv(B, BLK),),
        in_specs=[
            pl.BlockSpec((BLK,), lambda i: (i,)),
            pl.BlockSpec((ZMAXPAD, DIM), lambda i: (0, 0)),
        ],
        out_specs=pl.BlockSpec((BLK, DIM), lambda i: (i, 0)),
        out_shape=jax.ShapeDtypeStruct((B, DIM), jnp.float32),
    )(species, table)


def kernel(species, rand_encoding):
    return _encode(species, rand_encoding)
